# baseline (speedup 1.0000x reference)
import jax
import jax.numpy as jnp
from jax import lax
from jax.experimental import pallas as pl
from jax.experimental.pallas import tpu as pltpu
from jax.experimental.pallas import tpu_sc as plsc

D = 64
SEQ = 200
BATCH = 4096

_INFO = plsc.get_sparse_core_info()
NC, NS = _INFO.num_cores, _INFO.num_subcores
NW = NC * NS
SEQ_PER_W = BATCH // NW
SCALE = 1.0 / 8.0
NBUF = 4
NPAIR = SEQ_PER_W // NBUF
BP = BATCH // 2


def _gather_kernel(idx_hbm, table_hbm, out_hbm, idx_all, rows, gsema, gsemb, osem):
    wid = lax.axis_index("s") * NC + lax.axis_index("c")

    pltpu.sync_copy(idx_hbm.at[pl.ds(wid * SEQ_PER_W, SEQ_PER_W)], idx_all)

    def fire_gather(q, j):
        pltpu.make_async_copy(
            table_hbm.at[idx_all.at[q, pl.ds(0, 128)]],
            rows[j].at[pl.ds(0, 128)], gsema[j]).start()
        pltpu.make_async_copy(
            table_hbm.at[idx_all.at[q, pl.ds(128, 72)]],
            rows[j].at[pl.ds(128, 72)], gsemb[j]).start()

    def wait_gather(j):
        pltpu.make_async_copy(
            table_hbm.at[idx_all.at[0, pl.ds(0, 128)]],
            rows[j].at[pl.ds(0, 128)], gsema[j]).wait()
        pltpu.make_async_copy(
            table_hbm.at[idx_all.at[0, pl.ds(128, 72)]],
            rows[j].at[pl.ds(128, 72)], gsemb[j]).wait()

    def out_dst(q):
        bp = lax.rem(wid, NW // 2) * SEQ_PER_W + q
        h = lax.div(wid, NW // 2)
        return out_hbm.at[:, bp, pl.ds(h * D, D)]

    def fire_out(q, j):
        pltpu.make_async_copy(rows[j], out_dst(q), osem[j]).start()

    def wait_out(j):
        pltpu.make_async_copy(rows[j], out_dst(0), osem[j]).wait()

    fire_gather(0, 0)

    def pair(p, carry):
        for j in range(NBUF):
            q = p * NBUF + j
            nxt = (j + 1) % NBUF
            if j < NBUF - 1:
                @pl.when(p >= 1)
                def _():
                    wait_out(nxt)
                fire_gather(q + 1, nxt)
            else:
                wait_out(nxt)

                @pl.when(p < NPAIR - 1)
                def _():
                    fire_gather(q + 1, nxt)
            wait_gather(j)
            fire_out(q, j)
        return carry

    lax.fori_loop(0, NPAIR, pair, 0)
    for j in range(1, NBUF):
        wait_out(j)


def _finish_kernel(x_ref, pe_ref, sel_ref, eye_ref, out_ref):
    x = x_ref[...]
    pev = pe_ref[...]
    sel = sel_ref[...]
    eye = eye_ref[...]
    t = lax.dot_general(x, sel, (((2,), (0,)), ((), ())),
                        precision=lax.Precision.HIGHEST,
                        preferred_element_type=jnp.float32)
    u = lax.dot_general(t, eye, (((1,), (0,)), ((), ())),
                        precision=lax.Precision.HIGHEST,
                        preferred_element_type=jnp.float32)
    y = u * SCALE + pev[:, :, None]
    out_ref[...] = y.reshape(8 * D, 128)


@jax.jit
def kernel(input_seqs, table, pe):
    idx2d = input_seqs.astype(jnp.int32)
    raw = pl.kernel(
        _gather_kernel,
        out_type=jax.ShapeDtypeStruct((SEQ, BP, 2 * D), jnp.float32),
        mesh=plsc.VectorSubcoreMesh(core_axis_name="c", subcore_axis_name="s"),
        compiler_params=pltpu.CompilerParams(use_tc_tiling_on_sc=False),
        scratch_types=[
            pltpu.VMEM((SEQ_PER_W, SEQ), jnp.int32),
            [pltpu.VMEM((SEQ, D), jnp.float32)] * NBUF,
            [pltpu.SemaphoreType.DMA] * NBUF,
            [pltpu.SemaphoreType.DMA] * NBUF,
            [pltpu.SemaphoreType.DMA] * NBUF,
        ],
    )(idx2d, table)

    out2d = pl.pallas_call(
        _finish_kernel,
        out_shape=jax.ShapeDtypeStruct((SEQ * D, BATCH), jnp.float32),
        grid=(SEQ // 8, BATCH // 256, 2),
        in_specs=[
            pl.BlockSpec((8, 128, 2 * D), lambda i, k, h: (i, k, 0)),
            pl.BlockSpec((8, D), lambda i, k, h: (i, 0)),
            pl.BlockSpec((128, D), lambda i, k, h: (h, 0)),
            pl.BlockSpec((128, 128), lambda i, k, h: (0, 0)),
        ],
        out_specs=pl.BlockSpec((8 * D, 128), lambda i, k, h: (i, h * (BATCH // 256) + k)),
    )(raw, pe,
      jnp.concatenate([jnp.eye(128, D, dtype=jnp.float32),
                       jnp.eye(128, D, k=-D, dtype=jnp.float32)], axis=0),
      jnp.eye(128, dtype=jnp.float32))

    return out2d.reshape(SEQ, D, BATCH).transpose(2, 0, 1)


# speedup vs baseline: 1.3367x; 1.3367x over previous
import jax
import jax.numpy as jnp
from jax import lax
from jax.experimental import pallas as pl
from jax.experimental.pallas import tpu as pltpu
from jax.experimental.pallas import tpu_sc as plsc

VOCAB = 1000000
D = 64
SEQ = 200
BATCH = 4096

_INFO = plsc.get_sparse_core_info()
NC, NS = _INFO.num_cores, _INFO.num_subcores
NW = NC * NS
SEQ_PER_W = BATCH // NW
SCALE = 1.0 / 8.0
NBUF = 4
NPAIR = SEQ_PER_W // NBUF
BP = BATCH // 2


def _gather_kernel(idx_hbm, table_hbm, out_hbm, idx_all, rows, gsema, gsemb, osem):
    wid = lax.axis_index("s") * NC + lax.axis_index("c")

    pltpu.sync_copy(idx_hbm.at[pl.ds(wid * SEQ_PER_W, SEQ_PER_W)], idx_all)

    def fire_gather(q, j):
        pltpu.make_async_copy(
            table_hbm.at[idx_all.at[q, pl.ds(0, 128)]],
            rows[j].at[pl.ds(0, 128)], gsema[j]).start()
        pltpu.make_async_copy(
            table_hbm.at[idx_all.at[q, pl.ds(128, 72)]],
            rows[j].at[pl.ds(128, 72)], gsemb[j]).start()

    def wait_gather(j):
        pltpu.make_async_copy(
            table_hbm.at[idx_all.at[0, pl.ds(0, 128)]],
            rows[j].at[pl.ds(0, 128)], gsema[j]).wait()
        pltpu.make_async_copy(
            table_hbm.at[idx_all.at[0, pl.ds(128, 72)]],
            rows[j].at[pl.ds(128, 72)], gsemb[j]).wait()

    def out_dst(q):
        bp = lax.rem(wid, NW // 2) * SEQ_PER_W + q
        h = lax.div(wid, NW // 2)
        return out_hbm.at[:, bp, pl.ds(h * D, D)]

    def fire_out(q, j):
        pltpu.make_async_copy(rows[j], out_dst(q), osem[j]).start()

    def wait_out(j):
        pltpu.make_async_copy(rows[j], out_dst(0), osem[j]).wait()

    fire_gather(0, 0)

    def pair(p, carry):
        for j in range(NBUF):
            q = p * NBUF + j
            nxt = (j + 1) % NBUF
            if j < NBUF - 1:
                @pl.when(p >= 1)
                def _():
                    wait_out(nxt)
                fire_gather(q + 1, nxt)
            else:
                wait_out(nxt)

                @pl.when(p < NPAIR - 1)
                def _():
                    fire_gather(q + 1, nxt)
            wait_gather(j)
            fire_out(q, j)
        return carry

    lax.fori_loop(0, NPAIR, pair, 0)
    for j in range(1, NBUF):
        wait_out(j)


def _finish_kernel(x_ref, pe_ref, sel_ref, eye_ref, out_ref):
    x = x_ref[...]
    pev = pe_ref[...]
    sel = sel_ref[...]
    eye = eye_ref[...]
    t = lax.dot_general(x, sel, (((2,), (0,)), ((), ())),
                        preferred_element_type=jnp.float32)
    u = lax.dot_general(t, eye, (((1,), (0,)), ((), ())),
                        preferred_element_type=jnp.float32)
    y = u * SCALE + pev[:, :, None]
    out_ref[...] = y.reshape(8 * D, 128)


@jax.jit
def kernel(input_seqs, table, pe):
    idx2d = input_seqs.astype(jnp.int32) * 2
    table_rm = jnp.pad(table, ((0, 0), (0, D))).reshape(2 * VOCAB, D)
    raw = pl.kernel(
        _gather_kernel,
        out_type=jax.ShapeDtypeStruct((SEQ, BP, 2 * D), jnp.float32),
        mesh=plsc.VectorSubcoreMesh(core_axis_name="c", subcore_axis_name="s"),
        compiler_params=pltpu.CompilerParams(use_tc_tiling_on_sc=False),
        scratch_types=[
            pltpu.VMEM((SEQ_PER_W, SEQ), jnp.int32),
            [pltpu.VMEM((SEQ, D), jnp.float32)] * NBUF,
            [pltpu.SemaphoreType.DMA] * NBUF,
            [pltpu.SemaphoreType.DMA] * NBUF,
            [pltpu.SemaphoreType.DMA] * NBUF,
        ],
    )(idx2d, table_rm)

    out2d = pl.pallas_call(
        _finish_kernel,
        out_shape=jax.ShapeDtypeStruct((SEQ * D, BATCH), jnp.float32),
        grid=(SEQ // 8, BATCH // 256, 2),
        in_specs=[
            pl.BlockSpec((8, 128, 2 * D), lambda i, k, h: (i, k, 0)),
            pl.BlockSpec((8, D), lambda i, k, h: (i, 0)),
            pl.BlockSpec((128, D), lambda i, k, h: (h, 0)),
            pl.BlockSpec((128, 128), lambda i, k, h: (0, 0)),
        ],
        out_specs=pl.BlockSpec((8 * D, 128), lambda i, k, h: (i, h * (BATCH // 256) + k)),
    )(raw, pe,
      jnp.concatenate([jnp.eye(128, D, dtype=jnp.float32),
                       jnp.eye(128, D, k=-D, dtype=jnp.float32)], axis=0),
      jnp.eye(128, dtype=jnp.float32))

    return out2d.reshape(SEQ, D, BATCH).transpose(2, 0, 1)


# speedup vs baseline: 1.6554x; 1.2385x over previous
import jax
import jax.numpy as jnp
from jax import lax
from jax.experimental import pallas as pl
from jax.experimental.pallas import tpu as pltpu
from jax.experimental.pallas import tpu_sc as plsc

VOCAB = 1000000
D = 64
SEQ = 200
BATCH = 4096

_INFO = plsc.get_sparse_core_info()
NC, NS = _INFO.num_cores, _INFO.num_subcores
NW = NC * NS
SEQ_PER_W = BATCH // NW
SCALE = 1.0 / 8.0
NBUF = 4
NPAIR = SEQ_PER_W // NBUF
BP = BATCH // 2


def _gather_kernel(idx_hbm, table_hbm, out_hbm, idx_all, rows, gsema, gsemb, osem):
    wid = lax.axis_index("s") * NC + lax.axis_index("c")

    pltpu.sync_copy(idx_hbm.at[pl.ds(wid * SEQ_PER_W, SEQ_PER_W)], idx_all)

    def fire_gather(q, j):
        pltpu.make_async_copy(
            table_hbm.at[idx_all.at[q, pl.ds(0, 128)]],
            rows[j].at[pl.ds(0, 128)], gsema[j]).start()
        pltpu.make_async_copy(
            table_hbm.at[idx_all.at[q, pl.ds(128, 72)]],
            rows[j].at[pl.ds(128, 72)], gsemb[j]).start()

    def wait_gather(j):
        pltpu.make_async_copy(
            table_hbm.at[idx_all.at[0, pl.ds(0, 128)]],
            rows[j].at[pl.ds(0, 128)], gsema[j]).wait()
        pltpu.make_async_copy(
            table_hbm.at[idx_all.at[0, pl.ds(128, 72)]],
            rows[j].at[pl.ds(128, 72)], gsemb[j]).wait()

    def out_dst(q):
        bp = lax.rem(wid, NW // 2) * SEQ_PER_W + q
        h = lax.div(wid, NW // 2)
        return out_hbm.at[:, bp, pl.ds(h * D, D)]

    def fire_out(q, j):
        pltpu.make_async_copy(rows[j], out_dst(q), osem[j]).start()

    def wait_out(j):
        pltpu.make_async_copy(rows[j], out_dst(0), osem[j]).wait()

    fire_gather(0, 0)

    def pair(p, carry):
        for j in range(NBUF):
            q = p * NBUF + j
            nxt = (j + 1) % NBUF
            if j < NBUF - 1:
                @pl.when(p >= 1)
                def _():
                    wait_out(nxt)
                fire_gather(q + 1, nxt)
            else:
                wait_out(nxt)

                @pl.when(p < NPAIR - 1)
                def _():
                    fire_gather(q + 1, nxt)
            wait_gather(j)
            fire_out(q, j)
        return carry

    lax.fori_loop(0, NPAIR, pair, 0)
    for j in range(1, NBUF):
        wait_out(j)


def _finish_kernel(x_ref, pe_ref, sel_ref, out_ref):
    x = x_ref[...]
    pev = pe_ref[...]
    sel = sel_ref[...]
    u = lax.dot_general(sel, x, (((1,), (2,)), ((), ())),
                        preferred_element_type=jnp.float32)
    y = u.swapaxes(0, 1) * SCALE + pev[:, :, None]
    out_ref[...] = y.reshape(8 * D, 256)


@jax.jit
def kernel(input_seqs, table, pe):
    idx2d = input_seqs.astype(jnp.int32) * 2
    table_rm = jnp.pad(table, ((0, 0), (0, D))).reshape(2 * VOCAB, D)
    raw = pl.kernel(
        _gather_kernel,
        out_type=jax.ShapeDtypeStruct((SEQ, BP, 2 * D), jnp.float32),
        mesh=plsc.VectorSubcoreMesh(core_axis_name="c", subcore_axis_name="s"),
        compiler_params=pltpu.CompilerParams(use_tc_tiling_on_sc=False),
        scratch_types=[
            pltpu.VMEM((SEQ_PER_W, SEQ), jnp.int32),
            [pltpu.VMEM((SEQ, D), jnp.float32)] * NBUF,
            [pltpu.SemaphoreType.DMA] * NBUF,
            [pltpu.SemaphoreType.DMA] * NBUF,
            [pltpu.SemaphoreType.DMA] * NBUF,
        ],
    )(idx2d, table_rm)

    out2d = pl.pallas_call(
        _finish_kernel,
        out_shape=jax.ShapeDtypeStruct((SEQ * D, BATCH), jnp.float32),
        grid=(SEQ // 8, BATCH // 512, 2),
        in_specs=[
            pl.BlockSpec((8, 256, 2 * D), lambda i, k, h: (i, k, 0)),
            pl.BlockSpec((8, D), lambda i, k, h: (i, 0)),
            pl.BlockSpec((D, 128), lambda i, k, h: (h, 0)),
        ],
        out_specs=pl.BlockSpec((8 * D, 256), lambda i, k, h: (i, h * (BATCH // 512) + k)),
    )(raw, pe,
      jnp.concatenate([jnp.eye(D, 128, dtype=jnp.float32),
                       jnp.eye(D, 128, k=D, dtype=jnp.float32)], axis=0))

    return out2d.reshape(SEQ, D, BATCH).transpose(2, 0, 1)


# speedup vs baseline: 1.9856x; 1.1995x over previous
import jax
import jax.numpy as jnp
from jax import lax
from jax.experimental import pallas as pl
from jax.experimental.pallas import tpu as pltpu
from jax.experimental.pallas import tpu_sc as plsc

VOCAB = 1000000
D = 64
SEQ = 200
BATCH = 4096

_INFO = plsc.get_sparse_core_info()
NC, NS = _INFO.num_cores, _INFO.num_subcores
NW = NC * NS
SEQ_PER_W = BATCH // NW
SCALE = 1.0 / 8.0
NBUF = 4
NPAIR = SEQ_PER_W // NBUF
BP = BATCH // 2
SB = 40


def _gather_kernel(idx_hbm, table_hbm, out_hbm, idx_all, rows, gsema, gsemb, osem):
    wid = lax.axis_index("s") * NC + lax.axis_index("c")

    pltpu.sync_copy(idx_hbm.at[pl.ds(wid * SEQ_PER_W, SEQ_PER_W)], idx_all)

    def fire_gather(q, j):
        pltpu.make_async_copy(
            table_hbm.at[idx_all.at[q, pl.ds(0, 128)]],
            rows[j].at[pl.ds(0, 128)], gsema[j]).start()
        pltpu.make_async_copy(
            table_hbm.at[idx_all.at[q, pl.ds(128, 72)]],
            rows[j].at[pl.ds(128, 72)], gsemb[j]).start()

    def wait_gather(j):
        pltpu.make_async_copy(
            table_hbm.at[idx_all.at[0, pl.ds(0, 128)]],
            rows[j].at[pl.ds(0, 128)], gsema[j]).wait()
        pltpu.make_async_copy(
            table_hbm.at[idx_all.at[0, pl.ds(128, 72)]],
            rows[j].at[pl.ds(128, 72)], gsemb[j]).wait()

    def out_dst(q):
        bp = lax.rem(wid, NW // 2) * SEQ_PER_W + q
        h = lax.div(wid, NW // 2)
        return out_hbm.at[:, bp, pl.ds(h * D, D)]

    def fire_out(q, j):
        pltpu.make_async_copy(rows[j], out_dst(q), osem[j]).start()

    def wait_out(j):
        pltpu.make_async_copy(rows[j], out_dst(0), osem[j]).wait()

    fire_gather(0, 0)

    def pair(p, carry):
        for j in range(NBUF):
            q = p * NBUF + j
            nxt = (j + 1) % NBUF
            if j < NBUF - 1:
                @pl.when(p >= 1)
                def _():
                    wait_out(nxt)
                fire_gather(q + 1, nxt)
            else:
                wait_out(nxt)

                @pl.when(p < NPAIR - 1)
                def _():
                    fire_gather(q + 1, nxt)
            wait_gather(j)
            fire_out(q, j)
        return carry

    lax.fori_loop(0, NPAIR, pair, 0)
    for j in range(1, NBUF):
        wait_out(j)


def _finish_kernel(x_ref, pe_ref, sel_ref, out_ref):
    x = x_ref[...]
    pev = pe_ref[...]
    sel = sel_ref[...]
    u = lax.dot_general(sel, x, (((1,), (2,)), ((), ())),
                        preferred_element_type=jnp.float32)
    y = u.swapaxes(0, 1) * SCALE + pev[:, :, None]
    out_ref[...] = y.reshape(SB * D, 256)


@jax.jit
def kernel(input_seqs, table, pe):
    idx2d = input_seqs.astype(jnp.int32) * 2
    table_rm = jnp.pad(table, ((0, 0), (0, D))).reshape(2 * VOCAB, D)
    raw = pl.kernel(
        _gather_kernel,
        out_type=jax.ShapeDtypeStruct((SEQ, BP, 2 * D), jnp.float32),
        mesh=plsc.VectorSubcoreMesh(core_axis_name="c", subcore_axis_name="s"),
        compiler_params=pltpu.CompilerParams(use_tc_tiling_on_sc=False),
        scratch_types=[
            pltpu.VMEM((SEQ_PER_W, SEQ), jnp.int32),
            [pltpu.VMEM((SEQ, D), jnp.float32)] * NBUF,
            [pltpu.SemaphoreType.DMA] * NBUF,
            [pltpu.SemaphoreType.DMA] * NBUF,
            [pltpu.SemaphoreType.DMA] * NBUF,
        ],
    )(idx2d, table_rm)

    out2d = pl.pallas_call(
        _finish_kernel,
        out_shape=jax.ShapeDtypeStruct((SEQ * D, BATCH), jnp.float32),
        grid=(SEQ // SB, BATCH // 512, 2),
        in_specs=[
            pl.BlockSpec((SB, 256, 2 * D), lambda i, k, h: (i, k, 0)),
            pl.BlockSpec((SB, D), lambda i, k, h: (i, 0)),
            pl.BlockSpec((D, 128), lambda i, k, h: (h, 0)),
        ],
        out_specs=pl.BlockSpec((SB * D, 256), lambda i, k, h: (i, h * (BATCH // 512) + k)),
    )(raw, pe,
      jnp.concatenate([jnp.eye(D, 128, dtype=jnp.float32),
                       jnp.eye(D, 128, k=D, dtype=jnp.float32)], axis=0))

    return out2d.reshape(SEQ, D, BATCH).transpose(2, 0, 1)


# speedup vs baseline: 1.9872x; 1.0008x over previous
import jax
import jax.numpy as jnp
from jax import lax
from jax.experimental import pallas as pl
from jax.experimental.pallas import tpu as pltpu
from jax.experimental.pallas import tpu_sc as plsc

VOCAB = 1000000
D = 64
SEQ = 200
BATCH = 4096

_INFO = plsc.get_sparse_core_info()
NC, NS = _INFO.num_cores, _INFO.num_subcores
NW = NC * NS
SEQ_PER_W = BATCH // NW
SCALE = 1.0 / 8.0
NBUF = 4
NPAIR = SEQ_PER_W // NBUF
BP = BATCH // 2
SB = 40


def _gather_kernel(idx_hbm, table_hbm, out_hbm, idx_all, rows, gsema, gsemb, osem):
    wid = lax.axis_index("s") * NC + lax.axis_index("c")

    pltpu.sync_copy(idx_hbm.at[pl.ds(wid * SEQ_PER_W, SEQ_PER_W)], idx_all)

    def fire_gather(q, j):
        pltpu.make_async_copy(
            table_hbm.at[idx_all.at[q, pl.ds(0, 128)]],
            rows[j].at[pl.ds(0, 128)], gsema[j]).start()
        pltpu.make_async_copy(
            table_hbm.at[idx_all.at[q, pl.ds(128, 72)]],
            rows[j].at[pl.ds(128, 72)], gsemb[j]).start()

    def wait_gather(j):
        pltpu.make_async_copy(
            table_hbm.at[idx_all.at[0, pl.ds(0, 128)]],
            rows[j].at[pl.ds(0, 128)], gsema[j]).wait()
        pltpu.make_async_copy(
            table_hbm.at[idx_all.at[0, pl.ds(128, 72)]],
            rows[j].at[pl.ds(128, 72)], gsemb[j]).wait()

    def out_dst(q):
        bp = lax.rem(wid, NW // 2) * SEQ_PER_W + q
        h = lax.div(wid, NW // 2)
        return out_hbm.at[:, bp, pl.ds(h * D, D)]

    def fire_out(q, j):
        pltpu.make_async_copy(rows[j], out_dst(q), osem[j]).start()

    def wait_out(j):
        pltpu.make_async_copy(rows[j], out_dst(0), osem[j]).wait()

    fire_gather(0, 0)

    def pair(p, carry):
        for j in range(NBUF):
            q = p * NBUF + j
            nxt = (j + 1) % NBUF
            if j < NBUF - 1:
                @pl.when(p >= 1)
                def _():
                    wait_out(nxt)
                fire_gather(q + 1, nxt)
            else:
                wait_out(nxt)

                @pl.when(p < NPAIR - 1)
                def _():
                    fire_gather(q + 1, nxt)
            wait_gather(j)
            fire_out(q, j)
        return carry

    lax.fori_loop(0, NPAIR, pair, 0)
    for j in range(1, NBUF):
        wait_out(j)


def _finish_kernel(x_ref, pe_ref, sel_ref, out_ref):
    x = x_ref[...]
    pev = pe_ref[...]
    sel = sel_ref[...]
    u = lax.dot_general(sel, x, (((1,), (2,)), ((), ())),
                        preferred_element_type=jnp.float32)
    y = u.swapaxes(0, 1) * SCALE + pev[:, :, None]
    out_ref[...] = y.reshape(SB * D, 256)


@jax.jit
def kernel(input_seqs, table, pe):
    idx2d = input_seqs.astype(jnp.int32) * 2
    table_rm = jnp.pad(table, ((0, 0), (0, D))).reshape(2 * VOCAB, D)
    raw = pl.kernel(
        _gather_kernel,
        out_type=jax.ShapeDtypeStruct((SEQ, BP, 2 * D), jnp.float32),
        mesh=plsc.VectorSubcoreMesh(core_axis_name="c", subcore_axis_name="s"),
        compiler_params=pltpu.CompilerParams(use_tc_tiling_on_sc=False),
        scratch_types=[
            pltpu.VMEM((SEQ_PER_W, SEQ), jnp.int32),
            [pltpu.VMEM((SEQ, D), jnp.float32)] * NBUF,
            [pltpu.SemaphoreType.DMA] * NBUF,
            [pltpu.SemaphoreType.DMA] * NBUF,
            [pltpu.SemaphoreType.DMA] * NBUF,
        ],
    )(idx2d, table_rm)

    out2d = pl.pallas_call(
        _finish_kernel,
        out_shape=jax.ShapeDtypeStruct((SEQ * D, BATCH), jnp.float32),
        grid=(SEQ // SB, BATCH // 512, 2),
        in_specs=[
            pl.BlockSpec((SB, 256, 2 * D), lambda i, k, h: (i, k, 0)),
            pl.BlockSpec((SB, D), lambda i, k, h: (i, 0)),
            pl.BlockSpec((D, 128), lambda i, k, h: (h, 0)),
        ],
        out_specs=pl.BlockSpec((SB * D, 256), lambda i, k, h: (i, h * (BATCH // 512) + k)),
        compiler_params=pltpu.CompilerParams(
            dimension_semantics=("parallel", "parallel", "arbitrary")),
    )(raw, pe,
      jnp.concatenate([jnp.eye(D, 128, dtype=jnp.float32),
                       jnp.eye(D, 128, k=D, dtype=jnp.float32)], axis=0))

    return out2d.reshape(SEQ, D, BATCH).transpose(2, 0, 1)
